# baseline (speedup 1.0000x reference)
import functools

import jax
import jax.numpy as jnp
from jax import lax
from jax.experimental import pallas as pl
from jax.experimental.pallas import tpu as pltpu
from jax.experimental.pallas import tpu_sc as plsc

BATCH = 16384
D_ID = 128
D_REVIEW = 64
CHUNK = 128


def _build_kernel():
    info = plsc.get_sparse_core_info()
    num_cores = info.num_cores
    num_workers = num_cores * info.num_subcores
    b_per_w = BATCH // num_workers
    n_chunks = b_per_w // CHUNK

    mesh = plsc.VectorSubcoreMesh(core_axis_name="c", subcore_axis_name="s")

    @functools.partial(
        pl.kernel,
        mesh=mesh,
        compiler_params=pltpu.CompilerParams(use_tc_tiling_on_sc=False),
        out_type=[
            jax.ShapeDtypeStruct((BATCH, D_ID), jnp.float32),
            jax.ShapeDtypeStruct((BATCH, D_ID), jnp.float32),
            jax.ShapeDtypeStruct((BATCH, D_REVIEW), jnp.float32),
        ],
        scratch_types=[
            pltpu.VMEM((n_chunks, CHUNK), jnp.int32),
            pltpu.VMEM((CHUNK, D_ID), jnp.float32),
            pltpu.VMEM((CHUNK, D_ID), jnp.float32),
            pltpu.VMEM((CHUNK, D_REVIEW), jnp.float32),
            pltpu.VMEM((CHUNK, D_ID), jnp.float32),
            pltpu.VMEM((CHUNK, D_ID), jnp.float32),
            pltpu.VMEM((CHUNK, D_REVIEW), jnp.float32),
            pltpu.VMEM((CHUNK, D_ID), jnp.float32),
            pltpu.VMEM((CHUNK, D_ID), jnp.float32),
            pltpu.VMEM((CHUNK, D_REVIEW), jnp.float32),
            pltpu.SemaphoreType.DMA,
            pltpu.SemaphoreType.DMA,
            pltpu.SemaphoreType.DMA,
            pltpu.SemaphoreType.DMA,
            pltpu.SemaphoreType.DMA,
            pltpu.SemaphoreType.DMA,
        ],
    )
    def gather3(idx_hbm, protos_hbm, emb_hbm, review_hbm,
                proto_out, emb_out, review_out,
                idx_v, pv0, ev0, rv0, pv1, ev1, rv1, pv2, ev2, rv2,
                gs0, gs1, gs2, os0, os1, os2):
        wid = lax.axis_index("s") * num_cores + lax.axis_index("c")
        base = wid * b_per_w
        pltpu.sync_copy(idx_hbm.at[wid], idx_v)

        bufs = ((pv0, ev0, rv0), (pv1, ev1, rv1), (pv2, ev2, rv2))
        gsems = (gs0, gs1, gs2)
        osems = (os0, os1, os2)

        def start_gather(j, s):
            pv, ev, rv = bufs[s]
            row_idx = idx_v.at[j]
            return (
                pltpu.async_copy(protos_hbm.at[row_idx], pv, gsems[s]),
                pltpu.async_copy(emb_hbm.at[row_idx], ev, gsems[s]),
                pltpu.async_copy(review_hbm.at[row_idx], rv, gsems[s]),
            )

        def start_copyout(j, s):
            pv, ev, rv = bufs[s]
            off = base + j * CHUNK
            return (
                pltpu.async_copy(pv, proto_out.at[pl.ds(off, CHUNK)], osems[s]),
                pltpu.async_copy(ev, emb_out.at[pl.ds(off, CHUNK)], osems[s]),
                pltpu.async_copy(rv, review_out.at[pl.ds(off, CHUNK)], osems[s]),
            )

        nbuf = 3
        gather_h = [None] * nbuf
        copy_h = [None] * nbuf
        for j in range(min(nbuf, n_chunks)):
            gather_h[j] = start_gather(j, j)
        for j in range(n_chunks):
            s = j % nbuf
            if copy_h[s] is not None:
                for h in copy_h[s]:
                    h.wait()
                copy_h[s] = None
                gather_h[s] = start_gather(j, s)
            for h in gather_h[s]:
                h.wait()
            copy_h[s] = start_copyout(j, s)
        for s in range(nbuf):
            if copy_h[s] is not None:
                for h in copy_h[s]:
                    h.wait()

    return gather3, num_workers, n_chunks


def kernel(nodes_u, global_protos, u_emb_weight, u_review_weight):
    gather3, num_workers, n_chunks = _build_kernel()
    idx = nodes_u.astype(jnp.int32).reshape(num_workers, n_chunks, CHUNK)
    proto_feats, u_id_feats, u_review_feats = gather3(
        idx, global_protos, u_emb_weight, u_review_weight)
    return (proto_feats, u_id_feats, u_review_feats)
